# baseline (speedup 1.0000x reference)
import functools

import jax
import jax.numpy as jnp
from jax import lax
from jax.experimental import pallas as pl
from jax.experimental.pallas import tpu as pltpu
from jax.experimental.pallas import tpu_sc as plsc

_NC, _NS = 2, 16
_NW = _NC * _NS


def _gather3(embT, kggT, relT, scg_ids, kgg_ids, rel_ids, B, D):
    b_per_w = B // _NW
    mesh = plsc.VectorSubcoreMesh(core_axis_name="c", subcore_axis_name="s")

    @functools.partial(
        pl.kernel,
        mesh=mesh,
        compiler_params=pltpu.CompilerParams(
            use_tc_tiling_on_sc=False, needs_layout_passes=False),
        out_type=(
            jax.ShapeDtypeStruct((D, B), jnp.float32),
            jax.ShapeDtypeStruct((D, B), jnp.float32),
            jax.ShapeDtypeStruct((D, B), jnp.float32),
        ),
        scratch_types=[
            pltpu.VMEM((b_per_w,), jnp.int32),
            pltpu.VMEM((D, b_per_w), jnp.float32),
            pltpu.VMEM((D, D), jnp.float32),
            pltpu.SemaphoreType.DMA,
        ],
    )
    def k(embT_h, kggT_h, relT_h, scg_h, kggid_h, rid_h, o1, o2, o3,
          idx_v, obT_v, relall_v, sem):
        wid = lax.axis_index("s") * _NC + lax.axis_index("c")
        base = wid * b_per_w
        lanes = lax.iota(jnp.int32, 16)

        def do_big(tab_h, ids_h, out_h):
            pltpu.sync_copy(ids_h.at[pl.ds(base, b_per_w)], idx_v)
            for c in range(D):
                pltpu.async_copy(tab_h.at[c].at[idx_v], obT_v.at[c], sem)
            for c in range(D):
                pltpu.make_async_copy(tab_h.at[c].at[idx_v], obT_v.at[c],
                                      sem).wait()
            pltpu.sync_copy(obT_v, out_h.at[:, pl.ds(base, b_per_w)])

        do_big(embT_h, scg_h, o1)
        do_big(kggT_h, kggid_h, o2)

        pltpu.sync_copy(relT_h, relall_v)
        pltpu.sync_copy(rid_h.at[pl.ds(base, b_per_w)], idx_v)

        def rel_grp(g, carry):
            colv = g * 16 + lanes
            ridv = idx_v[pl.ds(g * 16, 16)]
            for c in range(D):
                cc = jnp.full((16,), c, jnp.int32)
                vals = plsc.load_gather(relall_v, [cc, ridv])
                plsc.store_scatter(obT_v, [cc, colv], vals)
            return carry

        lax.fori_loop(0, b_per_w // 16, rel_grp, 0)
        pltpu.sync_copy(obT_v, o3.at[:, pl.ds(base, b_per_w)])

    return k(embT, kggT, relT, scg_ids, kgg_ids, rel_ids)


def kernel(embedding, kgg_table, relation_table, scg_ids, relation_ids,
           kgg_ids):
    B, D = scg_ids.shape[0], embedding.shape[1]
    o1, o2, o3 = _gather3(
        embedding.T, kgg_table.T, relation_table.T,
        scg_ids.astype(jnp.int32), kgg_ids.astype(jnp.int32),
        relation_ids.astype(jnp.int32), B, D)
    return (o1.T, o2.T, o3.T)


# speedup vs baseline: 7.3265x; 7.3265x over previous
import functools

import jax
import jax.numpy as jnp
from jax import lax
from jax.experimental import pallas as pl
from jax.experimental.pallas import tpu as pltpu
from jax.experimental.pallas import tpu_sc as plsc

_NC, _NS = 2, 16
_NW = _NC * _NS


def _gather1(table, ids):
    B = ids.shape[0]
    D = table.shape[1]
    b_per_w = B // _NW
    mesh = plsc.VectorSubcoreMesh(core_axis_name="c", subcore_axis_name="s")

    @functools.partial(
        pl.kernel,
        mesh=mesh,
        compiler_params=pltpu.CompilerParams(use_tc_tiling_on_sc=False),
        out_type=jax.ShapeDtypeStruct((B, D), jnp.float32),
        scratch_types=[
            pltpu.VMEM((b_per_w,), jnp.int32),
            pltpu.VMEM((b_per_w, D), jnp.float32),
            pltpu.SemaphoreType.DMA,
        ],
    )
    def k(tab_h, ids_h, out_h, idx_v, rows_v, sem):
        wid = lax.axis_index("s") * _NC + lax.axis_index("c")
        base = wid * b_per_w
        pltpu.sync_copy(ids_h.at[pl.ds(base, b_per_w)], idx_v)
        pltpu.async_copy(tab_h.at[idx_v], rows_v, sem).wait()
        pltpu.sync_copy(rows_v, out_h.at[pl.ds(base, b_per_w)])

    return k(table, ids)


def kernel(embedding, kgg_table, relation_table, scg_ids, relation_ids,
           kgg_ids):
    return (
        _gather1(embedding, scg_ids.astype(jnp.int32)),
        _gather1(kgg_table, kgg_ids.astype(jnp.int32)),
        _gather1(relation_table, relation_ids.astype(jnp.int32)),
    )
